# Initial kernel scaffold; baseline (speedup 1.0000x reference)
import jax
import jax.numpy as jnp
from jax.experimental import pallas as pl


def kernel(h_one, e_e_i, e_e_j, W_qkv, b_qkv, ln1_scale, ln1_bias, W_mlp, b_mlp, ln2_scale, ln2_bias):
    raise NotImplementedError("write your pallas kernel here")


# speedup vs baseline: 1.0339x; 1.0339x over previous
import functools

import jax
import jax.numpy as jnp
from jax import lax
from jax.experimental import pallas as pl
from jax.experimental.pallas import tpu as pltpu

N = 10000
E = 160000
DIM = 256
HEADS = 8
HD = DIM // HEADS
HG = 2
HPG = HEADS // HG
GW = DIM // HG

ROWS_BLK = 1000
N_BLKS = N // ROWS_BLK


def _qkv_body(x_ref, w_ref, b_ref, q_ref, k_ref, v_ref):
    x = x_ref[...]
    qkv = jnp.dot(x, w_ref[...], preferred_element_type=jnp.float32) + b_ref[...]
    scale = 1.0 / (HD ** 0.5)
    q = qkv[:, :DIM] * scale
    k = qkv[:, DIM:2 * DIM]
    v = qkv[:, 2 * DIM:]
    q_ref[0] = q[:, :GW]
    q_ref[1] = q[:, GW:]
    k_ref[0] = k[:, :GW]
    k_ref[1] = k[:, GW:]
    v_ref[...] = v


def qkv_project(h_one, W_qkv, b_qkv):
    b2 = b_qkv.reshape(1, 3 * DIM)
    return pl.pallas_call(
        _qkv_body,
        grid=(N_BLKS,),
        in_specs=[
            pl.BlockSpec((ROWS_BLK, DIM), lambda i: (i, 0)),
            pl.BlockSpec((DIM, 3 * DIM), lambda i: (0, 0)),
            pl.BlockSpec((1, 3 * DIM), lambda i: (0, 0)),
        ],
        out_specs=[
            pl.BlockSpec((HG, ROWS_BLK, GW), lambda i: (0, i, 0)),
            pl.BlockSpec((HG, ROWS_BLK, GW), lambda i: (0, i, 0)),
            pl.BlockSpec((ROWS_BLK, DIM), lambda i: (i, 0)),
        ],
        out_shape=[
            jax.ShapeDtypeStruct((HG, N, GW), jnp.float32),
            jax.ShapeDtypeStruct((HG, N, GW), jnp.float32),
            jax.ShapeDtypeStruct((N, DIM), jnp.float32),
        ],
    )(h_one, W_qkv, b2)


def _final_body(h_ref, a0_ref, a1_ref, ln1s_ref, ln1b_ref, w_ref, b_ref,
                ln2s_ref, ln2b_ref, o_ref):
    h0 = h_ref[...]
    attn = jnp.concatenate([a0_ref[...], a1_ref[...]], axis=1)
    h = h0 + attn
    mean = jnp.mean(h, axis=-1, keepdims=True)
    var = jnp.mean((h - mean) ** 2, axis=-1, keepdims=True)
    h = (h - mean) * lax.rsqrt(var + 1e-6)
    h = h * ln1s_ref[...] + ln1b_ref[...]
    mlp = jnp.dot(h, w_ref[...], preferred_element_type=jnp.float32) + b_ref[...]
    mlp = jax.nn.gelu(mlp)
    h = h + mlp
    mean = jnp.mean(h, axis=-1, keepdims=True)
    var = jnp.mean((h - mean) ** 2, axis=-1, keepdims=True)
    h = (h - mean) * lax.rsqrt(var + 1e-6)
    o_ref[...] = h * ln2s_ref[...] + ln2b_ref[...]


def final_block(h_one, attn0, attn1, ln1_scale, ln1_bias, W_mlp, b_mlp,
                ln2_scale, ln2_bias):
    return pl.pallas_call(
        _final_body,
        grid=(N_BLKS,),
        in_specs=[
            pl.BlockSpec((ROWS_BLK, DIM), lambda i: (i, 0)),
            pl.BlockSpec((ROWS_BLK, GW), lambda i: (i, 0)),
            pl.BlockSpec((ROWS_BLK, GW), lambda i: (i, 0)),
            pl.BlockSpec((1, DIM), lambda i: (0, 0)),
            pl.BlockSpec((1, DIM), lambda i: (0, 0)),
            pl.BlockSpec((DIM, DIM), lambda i: (0, 0)),
            pl.BlockSpec((1, DIM), lambda i: (0, 0)),
            pl.BlockSpec((1, DIM), lambda i: (0, 0)),
            pl.BlockSpec((1, DIM), lambda i: (0, 0)),
        ],
        out_specs=pl.BlockSpec((ROWS_BLK, DIM), lambda i: (i, 0)),
        out_shape=jax.ShapeDtypeStruct((N, DIM), jnp.float32),
    )(h_one, attn0, attn1, ln1_scale.reshape(1, DIM), ln1_bias.reshape(1, DIM),
      W_mlp, b_mlp.reshape(1, DIM), ln2_scale.reshape(1, DIM),
      ln2_bias.reshape(1, DIM))


def _edge_pipeline(q2, k2, v, e_e_i, e_e_j):
    q = jnp.concatenate([q2[0], q2[1]], axis=1).reshape(N, HEADS, HD)
    k = jnp.concatenate([k2[0], k2[1]], axis=1).reshape(N, HEADS, HD)
    vv = v.reshape(N, HEADS, HD)
    logits = jnp.einsum('ehd,ehd->eh', q[e_e_i], k[e_e_j])
    ex = jnp.exp(logits)
    denom = jax.ops.segment_sum(ex, e_e_j, num_segments=N)
    w = ex / (denom[e_e_j] + 1e-30)
    attn = jax.ops.segment_sum(jnp.einsum('eh,ehd->ehd', w, vv[e_e_j]),
                               e_e_i, num_segments=N)
    attn = attn.reshape(N, DIM)
    return attn[:, :GW], attn[:, GW:]


def kernel(h_one, e_e_i, e_e_j, W_qkv, b_qkv, ln1_scale, ln1_bias, W_mlp,
           b_mlp, ln2_scale, ln2_bias):
    q2, k2, v = qkv_project(h_one, W_qkv, b_qkv)
    attn0, attn1 = _edge_pipeline(q2, k2, v, e_e_i, e_e_j)
    return final_block(h_one, attn0, attn1, ln1_scale, ln1_bias, W_mlp,
                       b_mlp, ln2_scale, ln2_bias)


# speedup vs baseline: 14.0621x; 13.6014x over previous
import functools

import jax
import jax.numpy as jnp
import numpy as np
from jax import lax
from jax.experimental import pallas as pl
from jax.experimental.pallas import tpu as pltpu
from jax.experimental.pallas import tpu_sc as plsc

N = 10000
E = 160000
DIM = 256
HEADS = 8
HD = DIM // HEADS
HG = 2
HPG = HEADS // HG
GW = DIM // HG

ROWS_BLK = 1000
N_BLKS = N // ROWS_BLK

NSC = 2
NTILES = 16
LANES = 16

EPT = E // NTILES
CH = 80
NCH = EPT // CH
GRP = CH // LANES
RT = 10
NSL = N // RT
OB = 200
DEN_W = 16

_SC_MESH = plsc.VectorSubcoreMesh(core_axis_name="c", subcore_axis_name="s",
                                  num_cores=NSC, num_subcores=NTILES)


def _qkv_body(x_ref, w_ref, b_ref, q_ref, k_ref, v_ref):
    x = x_ref[...]
    qkv = jnp.dot(x, w_ref[...], preferred_element_type=jnp.float32) + b_ref[...]
    scale = 1.0 / (HD ** 0.5)
    q = qkv[:, :DIM] * scale
    k = qkv[:, DIM:2 * DIM]
    v = qkv[:, 2 * DIM:]
    q_ref[0] = q[:, :GW]
    q_ref[1] = q[:, GW:]
    k_ref[0] = k[:, :GW]
    k_ref[1] = k[:, GW:]
    v_ref[...] = v


def qkv_project(h_one, W_qkv, b_qkv):
    b2 = b_qkv.reshape(1, 3 * DIM)
    return pl.pallas_call(
        _qkv_body,
        grid=(N_BLKS,),
        in_specs=[
            pl.BlockSpec((ROWS_BLK, DIM), lambda i: (i, 0)),
            pl.BlockSpec((DIM, 3 * DIM), lambda i: (0, 0)),
            pl.BlockSpec((1, 3 * DIM), lambda i: (0, 0)),
        ],
        out_specs=[
            pl.BlockSpec((HG, ROWS_BLK, GW), lambda i: (0, i, 0)),
            pl.BlockSpec((HG, ROWS_BLK, GW), lambda i: (0, i, 0)),
            pl.BlockSpec((ROWS_BLK, DIM), lambda i: (i, 0)),
        ],
        out_shape=[
            jax.ShapeDtypeStruct((HG, N, GW), jnp.float32),
            jax.ShapeDtypeStruct((HG, N, GW), jnp.float32),
            jax.ShapeDtypeStruct((N, DIM), jnp.float32),
        ],
    )(h_one, W_qkv, b2)


def _sc_pass_a(qlo_hbm, qhi_hbm, klo_hbm, khi_hbm, ei_hbm, ej_hbm,
               ex_hbm, den_hbm,
               idxi_all, idxj_all, qr0, kr0, qr1, kr1, exc_v, exd_v,
               zbuf_v, den_sp, sq0, sk0, sq1, sk1):
    c = lax.axis_index("c")
    s = lax.axis_index("s")
    cN = c * N
    base = s * EPT

    qrs, krs = (qr0, qr1), (kr0, kr1)
    sqs, sks = (sq0, sq1), (sk0, sk1)

    def zden(r, _):
        zbuf_v[r] = jnp.zeros((LANES,), jnp.float32)
        return 0
    lax.fori_loop(0, OB, zden, 0)

    @pl.when(s < RT)
    def _():
        for u in range(NSL // OB):
            off = pl.multiple_of(s * NSL + u * OB, 8)
            pltpu.sync_copy(zbuf_v, den_sp.at[pl.ds(off, OB)])

    def zex(r, _):
        exd_v[r] = jnp.zeros((LANES,), jnp.float32)
        return 0
    lax.fori_loop(0, CH, zex, 0)

    pltpu.sync_copy(ei_hbm.at[pl.ds(base, EPT)], idxi_all)
    pltpu.sync_copy(ej_hbm.at[pl.ds(base, EPT)], idxj_all)
    plsc.subcore_barrier()

    lane_iota = jnp.arange(LANES, dtype=jnp.int32)

    def start_g(g, p):
        isl = idxi_all.at[pl.ds(g * CH, CH)]
        jsl = idxj_all.at[pl.ds(g * CH, CH)]

        @pl.when(c == 0)
        def _():
            pltpu.async_copy(qlo_hbm.at[isl], qrs[p], sqs[p])
            pltpu.async_copy(klo_hbm.at[jsl], krs[p], sks[p])

        @pl.when(c == 1)
        def _():
            pltpu.async_copy(qhi_hbm.at[isl], qrs[p], sqs[p])
            pltpu.async_copy(khi_hbm.at[jsl], krs[p], sks[p])

    def wait_g(g, p):
        isl = idxi_all.at[pl.ds(g * CH, CH)]
        jsl = idxj_all.at[pl.ds(g * CH, CH)]

        @pl.when(c == 0)
        def _():
            pltpu.make_async_copy(qlo_hbm.at[isl], qrs[p], sqs[p]).wait()
            pltpu.make_async_copy(klo_hbm.at[jsl], krs[p], sks[p]).wait()

        @pl.when(c == 1)
        def _():
            pltpu.make_async_copy(qhi_hbm.at[isl], qrs[p], sqs[p]).wait()
            pltpu.make_async_copy(khi_hbm.at[jsl], krs[p], sks[p]).wait()

    def compute(p):
        qr, kr = qrs[p], krs[p]

        def grp(t, _):
            rows = t * LANES + lane_iota
            for h in range(HPG):
                acc = jnp.zeros((LANES,), jnp.float32)
                for dd in range(HD):
                    col = jnp.full((LANES,), h * HD + dd, jnp.int32)
                    qv = plsc.load_gather(qr, [rows, col])
                    kv = plsc.load_gather(kr, [rows, col])
                    acc = acc + qv * kv
                exh = jnp.exp(acc)
                plsc.store_scatter(exc_v, [rows * HPG + h], exh)
                plsc.store_scatter(exd_v, [rows, jnp.full((LANES,), h,
                                                          jnp.int32)], exh)
            return 0
        lax.fori_loop(0, GRP, grp, 0)

    def writes(g):
        e0 = base + g * CH
        pltpu.sync_copy(exc_v, ex_hbm.at[pl.ds((c * E + e0) * HPG, CH * HPG)])
        pltpu.sync_copy(exd_v, den_sp.at[idxj_all.at[pl.ds(g * CH, CH)]],
                        add=True)

    start_g(0, 0)

    def pair(i, _):
        c0 = 2 * i
        start_g(c0 + 1, 1)
        wait_g(c0, 0)
        compute(0)
        start_g(c0 + 2, 0)
        writes(c0)
        wait_g(c0 + 1, 1)
        compute(1)
        writes(c0 + 1)
        return 0
    lax.fori_loop(0, (NCH - 1) // 2, pair, 0)

    wait_g(NCH - 1, 0)
    compute(0)
    writes(NCH - 1)

    plsc.subcore_barrier()

    @pl.when(s < RT)
    def _():
        for u in range(NSL // OB):
            off = pl.multiple_of(s * NSL + u * OB, 8)
            offh = pl.multiple_of(cN + s * NSL + u * OB, 8)
            pltpu.sync_copy(den_sp.at[pl.ds(off, OB)],
                            den_hbm.at[pl.ds(offh, OB)])


def sc_pass_a(q2, k2, e_e_i, e_e_j):
    f = pl.kernel(
        _sc_pass_a,
        out_type=[
            jax.ShapeDtypeStruct((HG * E * HPG,), jnp.float32),
            jax.ShapeDtypeStruct((HG * N, DEN_W), jnp.float32),
        ],
        mesh=_SC_MESH,
        compiler_params=pltpu.CompilerParams(needs_layout_passes=False,
                                             use_tc_tiling_on_sc=False),
        scratch_types=[
            pltpu.VMEM((EPT,), jnp.int32),
            pltpu.VMEM((EPT,), jnp.int32),
            pltpu.VMEM((CH, GW), jnp.float32),
            pltpu.VMEM((CH, GW), jnp.float32),
            pltpu.VMEM((CH, GW), jnp.float32),
            pltpu.VMEM((CH, GW), jnp.float32),
            pltpu.VMEM((CH * HPG,), jnp.float32),
            pltpu.VMEM((CH, DEN_W), jnp.float32),
            pltpu.VMEM((OB, DEN_W), jnp.float32),
            pltpu.VMEM_SHARED((N, DEN_W), jnp.float32),
            pltpu.SemaphoreType.DMA,
            pltpu.SemaphoreType.DMA,
            pltpu.SemaphoreType.DMA,
            pltpu.SemaphoreType.DMA,
        ],
    )
    return f(q2[0], q2[1], k2[0], k2[1], e_e_i, e_e_j)


_BCAST = np.zeros((DEN_W, GW), dtype=np.float32)
for _h in range(HPG):
    _BCAST[_h, _h * HD:(_h + 1) * HD] = 1.0


def _vn_body(v_ref, den_ref, b_ref, o_ref):
    recip = 1.0 / (den_ref[...] + 1e-30)
    rep = jnp.dot(recip, b_ref[...], preferred_element_type=jnp.float32)
    o_ref[...] = v_ref[...] * rep


def vnormalize(v, den):
    return pl.pallas_call(
        _vn_body,
        grid=(HG, N_BLKS),
        in_specs=[
            pl.BlockSpec((ROWS_BLK, GW), lambda i, j: (j, i)),
            pl.BlockSpec((ROWS_BLK, DEN_W), lambda i, j: (i * N_BLKS + j, 0)),
            pl.BlockSpec((DEN_W, GW), lambda i, j: (0, 0)),
        ],
        out_specs=pl.BlockSpec((ROWS_BLK, GW), lambda i, j: (i * N_BLKS + j, 0)),
        out_shape=jax.ShapeDtypeStruct((HG * N, GW), jnp.float32),
    )(v, den, jnp.asarray(_BCAST))


def _sc_pass_b(vnlo_hbm, vnhi_hbm, ei_hbm, ej_hbm, ex_hbm, attn_hbm,
               idxi_all, idxj_all, vr0, vr1, exc0, exc1,
               acc_sp, sv0, sv1, se0, se1):
    c = lax.axis_index("c")
    s = lax.axis_index("s")
    cN = c * N
    base = s * EPT

    vrs, excs = (vr0, vr1), (exc0, exc1)
    svs, ses = (sv0, sv1), (se0, se1)

    def zrow(r, _):
        for u in range(GW // LANES):
            vr0[r, pl.ds(u * LANES, LANES)] = jnp.zeros((LANES,), jnp.float32)
        return 0
    lax.fori_loop(0, CH, zrow, 0)

    @pl.when(s < RT)
    def _():
        for u in range(NSL // 40):
            off = pl.multiple_of(s * NSL + u * 40, 8)
            pltpu.sync_copy(vr0.at[pl.ds(0, 40)], acc_sp.at[pl.ds(off, 40)])

    pltpu.sync_copy(ei_hbm.at[pl.ds(base, EPT)], idxi_all)
    pltpu.sync_copy(ej_hbm.at[pl.ds(base, EPT)], idxj_all)
    plsc.subcore_barrier()

    lane_iota = jnp.arange(LANES, dtype=jnp.int32)

    def start_g(g, p):
        jsl = idxj_all.at[pl.ds(g * CH, CH)]
        e0 = base + g * CH
        pltpu.async_copy(ex_hbm.at[pl.ds((c * E + e0) * HPG, CH * HPG)],
                         excs[p], ses[p])

        @pl.when(c == 0)
        def _():
            pltpu.async_copy(vnlo_hbm.at[jsl], vrs[p], svs[p])

        @pl.when(c == 1)
        def _():
            pltpu.async_copy(vnhi_hbm.at[jsl], vrs[p], svs[p])

    def wait_g(g, p):
        jsl = idxj_all.at[pl.ds(g * CH, CH)]
        e0 = base + g * CH
        pltpu.make_async_copy(
            ex_hbm.at[pl.ds((c * E + e0) * HPG, CH * HPG)],
            excs[p], ses[p]).wait()

        @pl.when(c == 0)
        def _():
            pltpu.make_async_copy(vnlo_hbm.at[jsl], vrs[p], svs[p]).wait()

        @pl.when(c == 1)
        def _():
            pltpu.make_async_copy(vnhi_hbm.at[jsl], vrs[p], svs[p]).wait()

    def compute(p):
        vr, exc = vrs[p], excs[p]

        epg = LANES // HPG

        def egrp(g, _):
            exv = exc[pl.ds(g * LANES, LANES)]
            for sub in range(epg):
                e = g * epg + sub
                for h in range(HPG):
                    wv = jnp.full((LANES,), exv[sub * HPG + h])
                    for u in range(HD // LANES):
                        cs = pl.ds(h * HD + u * LANES, LANES)
                        vr[e, cs] = vr[e, cs] * wv
            return 0
        lax.fori_loop(0, CH // epg, egrp, 0)

    def scatter(g, p):
        pltpu.sync_copy(vrs[p], acc_sp.at[idxi_all.at[pl.ds(g * CH, CH)]],
                        add=True)

    start_g(0, 0)

    def pair(i, _):
        c0 = 2 * i
        start_g(c0 + 1, 1)
        wait_g(c0, 0)
        compute(0)
        scatter(c0, 0)
        start_g(c0 + 2, 0)
        wait_g(c0 + 1, 1)
        compute(1)
        scatter(c0 + 1, 1)
        return 0
    lax.fori_loop(0, (NCH - 1) // 2, pair, 0)

    wait_g(NCH - 1, 0)
    compute(0)
    scatter(NCH - 1, 0)

    plsc.subcore_barrier()

    @pl.when(s < RT)
    def _():
        for u in range(NSL // OB):
            off = pl.multiple_of(s * NSL + u * OB, 8)
            offh = pl.multiple_of(cN + s * NSL + u * OB, 8)
            pltpu.sync_copy(acc_sp.at[pl.ds(off, OB)],
                            attn_hbm.at[pl.ds(offh, OB)])


def sc_pass_b(vn, e_e_i, e_e_j, ex):
    f = pl.kernel(
        _sc_pass_b,
        out_type=jax.ShapeDtypeStruct((HG * N, GW), jnp.float32),
        mesh=_SC_MESH,
        compiler_params=pltpu.CompilerParams(needs_layout_passes=False,
                                             use_tc_tiling_on_sc=False),
        scratch_types=[
            pltpu.VMEM((EPT,), jnp.int32),
            pltpu.VMEM((EPT,), jnp.int32),
            pltpu.VMEM((CH, GW), jnp.float32),
            pltpu.VMEM((CH, GW), jnp.float32),
            pltpu.VMEM((CH * HPG,), jnp.float32),
            pltpu.VMEM((CH * HPG,), jnp.float32),
            pltpu.VMEM_SHARED((N, GW), jnp.float32),
            pltpu.SemaphoreType.DMA,
            pltpu.SemaphoreType.DMA,
            pltpu.SemaphoreType.DMA,
            pltpu.SemaphoreType.DMA,
        ],
    )
    return f(vn[:N], vn[N:], e_e_i, e_e_j, ex)


def _final_body(h_ref, a0_ref, a1_ref, ln1s_ref, ln1b_ref, w_ref, b_ref,
                ln2s_ref, ln2b_ref, o_ref):
    h0 = h_ref[...]
    attn = jnp.concatenate([a0_ref[...], a1_ref[...]], axis=1)
    h = h0 + attn
    mean = jnp.mean(h, axis=-1, keepdims=True)
    var = jnp.mean((h - mean) ** 2, axis=-1, keepdims=True)
    h = (h - mean) * lax.rsqrt(var + 1e-6)
    h = h * ln1s_ref[...] + ln1b_ref[...]
    mlp = jnp.dot(h, w_ref[...], preferred_element_type=jnp.float32) + b_ref[...]
    mlp = jax.nn.gelu(mlp)
    h = h + mlp
    mean = jnp.mean(h, axis=-1, keepdims=True)
    var = jnp.mean((h - mean) ** 2, axis=-1, keepdims=True)
    h = (h - mean) * lax.rsqrt(var + 1e-6)
    o_ref[...] = h * ln2s_ref[...] + ln2b_ref[...]


def final_block(h_one, attn0, attn1, ln1_scale, ln1_bias, W_mlp, b_mlp,
                ln2_scale, ln2_bias):
    return pl.pallas_call(
        _final_body,
        grid=(N_BLKS,),
        in_specs=[
            pl.BlockSpec((ROWS_BLK, DIM), lambda i: (i, 0)),
            pl.BlockSpec((ROWS_BLK, GW), lambda i: (i, 0)),
            pl.BlockSpec((ROWS_BLK, GW), lambda i: (i, 0)),
            pl.BlockSpec((1, DIM), lambda i: (0, 0)),
            pl.BlockSpec((1, DIM), lambda i: (0, 0)),
            pl.BlockSpec((DIM, DIM), lambda i: (0, 0)),
            pl.BlockSpec((1, DIM), lambda i: (0, 0)),
            pl.BlockSpec((1, DIM), lambda i: (0, 0)),
            pl.BlockSpec((1, DIM), lambda i: (0, 0)),
        ],
        out_specs=pl.BlockSpec((ROWS_BLK, DIM), lambda i: (i, 0)),
        out_shape=jax.ShapeDtypeStruct((N, DIM), jnp.float32),
    )(h_one, attn0, attn1, ln1_scale.reshape(1, DIM), ln1_bias.reshape(1, DIM),
      W_mlp, b_mlp.reshape(1, DIM), ln2_scale.reshape(1, DIM),
      ln2_bias.reshape(1, DIM))


def kernel(h_one, e_e_i, e_e_j, W_qkv, b_qkv, ln1_scale, ln1_bias, W_mlp,
           b_mlp, ln2_scale, ln2_bias):
    e_e_i = e_e_i.astype(jnp.int32)
    e_e_j = e_e_j.astype(jnp.int32)
    q2, k2, v = qkv_project(h_one, W_qkv, b_qkv)
    ex, den = sc_pass_a(q2, k2, e_e_i, e_e_j)
    vn = vnormalize(v, den)
    attn = sc_pass_b(vn, e_e_i, e_e_j, ex)
    return final_block(h_one, attn[:N], attn[N:], ln1_scale, ln1_bias,
                       W_mlp, b_mlp, ln2_scale, ln2_bias)


# speedup vs baseline: 47.7200x; 3.3935x over previous
import functools

import jax
import jax.numpy as jnp
import numpy as np
from jax import lax
from jax.experimental import pallas as pl
from jax.experimental.pallas import tpu as pltpu
from jax.experimental.pallas import tpu_sc as plsc

N = 10000
E = 160000
DIM = 256
HEADS = 8
HD = DIM // HEADS
HG = 2
HPG = HEADS // HG
GW = DIM // HG

ROWS_BLK = 1000
N_BLKS = N // ROWS_BLK

NSC = 2
NTILES = 16
LANES = 16

EPT = E // NTILES
CH = 80
NCH = EPT // CH
GRP = CH // LANES
RT = 10
NSL = N // RT
OB = 200
DEN_W = 16

_SC_MESH = plsc.VectorSubcoreMesh(core_axis_name="c", subcore_axis_name="s",
                                  num_cores=NSC, num_subcores=NTILES)


def _qkv_body(x_ref, w_ref, b_ref, q_ref, k_ref, v_ref):
    x = x_ref[...]
    qkv = jnp.dot(x, w_ref[...], preferred_element_type=jnp.float32) + b_ref[...]
    scale = 1.0 / (HD ** 0.5)
    q = qkv[:, :DIM] * scale
    k = qkv[:, DIM:2 * DIM]
    v = qkv[:, 2 * DIM:]
    q_ref[0] = q[:, :GW]
    q_ref[1] = q[:, GW:]
    k_ref[0] = k[:, :GW]
    k_ref[1] = k[:, GW:]
    v_ref[...] = v


def qkv_project(h_one, W_qkv, b_qkv):
    b2 = b_qkv.reshape(1, 3 * DIM)
    return pl.pallas_call(
        _qkv_body,
        grid=(N_BLKS,),
        in_specs=[
            pl.BlockSpec((ROWS_BLK, DIM), lambda i: (i, 0)),
            pl.BlockSpec((DIM, 3 * DIM), lambda i: (0, 0)),
            pl.BlockSpec((1, 3 * DIM), lambda i: (0, 0)),
        ],
        out_specs=[
            pl.BlockSpec((HG, ROWS_BLK, GW), lambda i: (0, i, 0)),
            pl.BlockSpec((HG, ROWS_BLK, GW), lambda i: (0, i, 0)),
            pl.BlockSpec((ROWS_BLK, DIM), lambda i: (i, 0)),
        ],
        out_shape=[
            jax.ShapeDtypeStruct((HG, N, GW), jnp.float32),
            jax.ShapeDtypeStruct((HG, N, GW), jnp.float32),
            jax.ShapeDtypeStruct((N, DIM), jnp.float32),
        ],
    )(h_one, W_qkv, b2)


def _sc_pass_a(qlo_hbm, qhi_hbm, klo_hbm, khi_hbm, ei_hbm, ej_hbm,
               ex_hbm, den_hbm,
               idxi_all, idxj_all, qr0, kr0, qr1, kr1, exc_v, exd_v,
               zbuf_v, den_sp, sq0, sk0, sq1, sk1):
    c = lax.axis_index("c")
    s = lax.axis_index("s")
    cN = c * N
    base = s * EPT

    qrs, krs = (qr0, qr1), (kr0, kr1)
    sqs, sks = (sq0, sq1), (sk0, sk1)

    def zden(r, _):
        zbuf_v[r] = jnp.zeros((LANES,), jnp.float32)
        return 0
    lax.fori_loop(0, OB, zden, 0)

    @pl.when(s < RT)
    def _():
        for u in range(NSL // OB):
            off = pl.multiple_of(s * NSL + u * OB, 8)
            pltpu.sync_copy(zbuf_v, den_sp.at[pl.ds(off, OB)])

    def zex(r, _):
        exd_v[r] = jnp.zeros((LANES,), jnp.float32)
        return 0
    lax.fori_loop(0, CH, zex, 0)

    pltpu.sync_copy(ei_hbm.at[pl.ds(base, EPT)], idxi_all)
    pltpu.sync_copy(ej_hbm.at[pl.ds(base, EPT)], idxj_all)
    plsc.subcore_barrier()

    lane_iota = jnp.arange(LANES, dtype=jnp.int32)

    def start_g(g, p):
        isl = idxi_all.at[pl.ds(g * CH, CH)]
        jsl = idxj_all.at[pl.ds(g * CH, CH)]

        @pl.when(c == 0)
        def _():
            pltpu.async_copy(qlo_hbm.at[isl], qrs[p], sqs[p])
            pltpu.async_copy(klo_hbm.at[jsl], krs[p], sks[p])

        @pl.when(c == 1)
        def _():
            pltpu.async_copy(qhi_hbm.at[isl], qrs[p], sqs[p])
            pltpu.async_copy(khi_hbm.at[jsl], krs[p], sks[p])

    def wait_g(g, p):
        isl = idxi_all.at[pl.ds(g * CH, CH)]
        jsl = idxj_all.at[pl.ds(g * CH, CH)]

        @pl.when(c == 0)
        def _():
            pltpu.make_async_copy(qlo_hbm.at[isl], qrs[p], sqs[p]).wait()
            pltpu.make_async_copy(klo_hbm.at[jsl], krs[p], sks[p]).wait()

        @pl.when(c == 1)
        def _():
            pltpu.make_async_copy(qhi_hbm.at[isl], qrs[p], sqs[p]).wait()
            pltpu.make_async_copy(khi_hbm.at[jsl], krs[p], sks[p]).wait()

    def compute(p):
        qr, kr = qrs[p], krs[p]
        epg = LANES // HPG

        def egrp(g, _):
            z = jnp.zeros((LANES,), jnp.float32)
            for sub in range(epg):
                e = g * epg + sub
                for h in range(HPG):
                    a = (qr[e, pl.ds(h * HD, LANES)]
                         * kr[e, pl.ds(h * HD, LANES)]
                         + qr[e, pl.ds(h * HD + LANES, LANES)]
                         * kr[e, pl.ds(h * HD + LANES, LANES)])
                    for kk in (8, 4, 2, 1):
                        a = a + a.at[lane_iota ^ kk].get(
                            mode="promise_in_bounds")
                    z = jnp.where(lane_iota == sub * HPG + h, a, z)
            ev = jnp.exp(z)
            exc_v[pl.ds(g * LANES, LANES)] = ev
            for sub in range(epg):
                row = ev.at[(lane_iota & (HPG - 1)) + sub * HPG].get(
                    mode="promise_in_bounds")
                exd_v[g * epg + sub, pl.ds(0, DEN_W)] = jnp.where(
                    lane_iota < HPG, row, 0.0)
            return 0
        lax.fori_loop(0, CH // epg, egrp, 0)

    def writes(g):
        e0 = base + g * CH
        pltpu.sync_copy(exc_v, ex_hbm.at[pl.ds((c * E + e0) * HPG, CH * HPG)])
        pltpu.sync_copy(exd_v, den_sp.at[idxj_all.at[pl.ds(g * CH, CH)]],
                        add=True)

    start_g(0, 0)

    def pair(i, _):
        c0 = 2 * i
        start_g(c0 + 1, 1)
        wait_g(c0, 0)
        compute(0)
        start_g(c0 + 2, 0)
        writes(c0)
        wait_g(c0 + 1, 1)
        compute(1)
        writes(c0 + 1)
        return 0
    lax.fori_loop(0, (NCH - 1) // 2, pair, 0)

    wait_g(NCH - 1, 0)
    compute(0)
    writes(NCH - 1)

    plsc.subcore_barrier()

    @pl.when(s < RT)
    def _():
        for u in range(NSL // OB):
            off = pl.multiple_of(s * NSL + u * OB, 8)
            offh = pl.multiple_of(cN + s * NSL + u * OB, 8)
            pltpu.sync_copy(den_sp.at[pl.ds(off, OB)],
                            den_hbm.at[pl.ds(offh, OB)])


def sc_pass_a(q2, k2, e_e_i, e_e_j):
    f = pl.kernel(
        _sc_pass_a,
        out_type=[
            jax.ShapeDtypeStruct((HG * E * HPG,), jnp.float32),
            jax.ShapeDtypeStruct((HG * N, DEN_W), jnp.float32),
        ],
        mesh=_SC_MESH,
        compiler_params=pltpu.CompilerParams(needs_layout_passes=False,
                                             use_tc_tiling_on_sc=False),
        scratch_types=[
            pltpu.VMEM((EPT,), jnp.int32),
            pltpu.VMEM((EPT,), jnp.int32),
            pltpu.VMEM((CH, GW), jnp.float32),
            pltpu.VMEM((CH, GW), jnp.float32),
            pltpu.VMEM((CH, GW), jnp.float32),
            pltpu.VMEM((CH, GW), jnp.float32),
            pltpu.VMEM((CH * HPG,), jnp.float32),
            pltpu.VMEM((CH, DEN_W), jnp.float32),
            pltpu.VMEM((OB, DEN_W), jnp.float32),
            pltpu.VMEM_SHARED((N, DEN_W), jnp.float32),
            pltpu.SemaphoreType.DMA,
            pltpu.SemaphoreType.DMA,
            pltpu.SemaphoreType.DMA,
            pltpu.SemaphoreType.DMA,
        ],
    )
    return f(q2[0], q2[1], k2[0], k2[1], e_e_i, e_e_j)


_BCAST = np.zeros((DEN_W, GW), dtype=np.float32)
for _h in range(HPG):
    _BCAST[_h, _h * HD:(_h + 1) * HD] = 1.0


def _vn_body(v_ref, den_ref, b_ref, o_ref):
    recip = 1.0 / (den_ref[...] + 1e-30)
    rep = jnp.dot(recip, b_ref[...], preferred_element_type=jnp.float32)
    o_ref[...] = v_ref[...] * rep


def vnormalize(v, den):
    return pl.pallas_call(
        _vn_body,
        grid=(HG, N_BLKS),
        in_specs=[
            pl.BlockSpec((ROWS_BLK, GW), lambda i, j: (j, i)),
            pl.BlockSpec((ROWS_BLK, DEN_W), lambda i, j: (i * N_BLKS + j, 0)),
            pl.BlockSpec((DEN_W, GW), lambda i, j: (0, 0)),
        ],
        out_specs=pl.BlockSpec((ROWS_BLK, GW), lambda i, j: (i * N_BLKS + j, 0)),
        out_shape=jax.ShapeDtypeStruct((HG * N, GW), jnp.float32),
    )(v, den, jnp.asarray(_BCAST))


def _sc_pass_b(vnlo_hbm, vnhi_hbm, ei_hbm, ej_hbm, ex_hbm, attn_hbm,
               idxi_all, idxj_all, vr0, vr1, exc0, exc1,
               acc_sp, sv0, sv1, se0, se1):
    c = lax.axis_index("c")
    s = lax.axis_index("s")
    cN = c * N
    base = s * EPT

    vrs, excs = (vr0, vr1), (exc0, exc1)
    svs, ses = (sv0, sv1), (se0, se1)

    def zrow(r, _):
        for u in range(GW // LANES):
            vr0[r, pl.ds(u * LANES, LANES)] = jnp.zeros((LANES,), jnp.float32)
        return 0
    lax.fori_loop(0, CH, zrow, 0)

    @pl.when(s < RT)
    def _():
        for u in range(NSL // 40):
            off = pl.multiple_of(s * NSL + u * 40, 8)
            pltpu.sync_copy(vr0.at[pl.ds(0, 40)], acc_sp.at[pl.ds(off, 40)])

    pltpu.sync_copy(ei_hbm.at[pl.ds(base, EPT)], idxi_all)
    pltpu.sync_copy(ej_hbm.at[pl.ds(base, EPT)], idxj_all)
    plsc.subcore_barrier()

    lane_iota = jnp.arange(LANES, dtype=jnp.int32)

    def start_g(g, p):
        jsl = idxj_all.at[pl.ds(g * CH, CH)]
        e0 = base + g * CH
        pltpu.async_copy(ex_hbm.at[pl.ds((c * E + e0) * HPG, CH * HPG)],
                         excs[p], ses[p])

        @pl.when(c == 0)
        def _():
            pltpu.async_copy(vnlo_hbm.at[jsl], vrs[p], svs[p])

        @pl.when(c == 1)
        def _():
            pltpu.async_copy(vnhi_hbm.at[jsl], vrs[p], svs[p])

    def wait_g(g, p):
        jsl = idxj_all.at[pl.ds(g * CH, CH)]
        e0 = base + g * CH
        pltpu.make_async_copy(
            ex_hbm.at[pl.ds((c * E + e0) * HPG, CH * HPG)],
            excs[p], ses[p]).wait()

        @pl.when(c == 0)
        def _():
            pltpu.make_async_copy(vnlo_hbm.at[jsl], vrs[p], svs[p]).wait()

        @pl.when(c == 1)
        def _():
            pltpu.make_async_copy(vnhi_hbm.at[jsl], vrs[p], svs[p]).wait()

    def compute(p):
        vr, exc = vrs[p], excs[p]

        epg = LANES // HPG

        def egrp(g, _):
            exv = exc[pl.ds(g * LANES, LANES)]
            for sub in range(epg):
                e = g * epg + sub
                for h in range(HPG):
                    wv = jnp.full((LANES,), exv[sub * HPG + h])
                    for u in range(HD // LANES):
                        cs = pl.ds(h * HD + u * LANES, LANES)
                        vr[e, cs] = vr[e, cs] * wv
            return 0
        lax.fori_loop(0, CH // epg, egrp, 0)

    def scatter(g, p):
        pltpu.sync_copy(vrs[p], acc_sp.at[idxi_all.at[pl.ds(g * CH, CH)]],
                        add=True)

    start_g(0, 0)

    def pair(i, _):
        c0 = 2 * i
        start_g(c0 + 1, 1)
        wait_g(c0, 0)
        compute(0)
        scatter(c0, 0)
        start_g(c0 + 2, 0)
        wait_g(c0 + 1, 1)
        compute(1)
        scatter(c0 + 1, 1)
        return 0
    lax.fori_loop(0, (NCH - 1) // 2, pair, 0)

    wait_g(NCH - 1, 0)
    compute(0)
    scatter(NCH - 1, 0)

    plsc.subcore_barrier()

    @pl.when(s < RT)
    def _():
        for u in range(NSL // OB):
            off = pl.multiple_of(s * NSL + u * OB, 8)
            offh = pl.multiple_of(cN + s * NSL + u * OB, 8)
            pltpu.sync_copy(acc_sp.at[pl.ds(off, OB)],
                            attn_hbm.at[pl.ds(offh, OB)])


def sc_pass_b(vn, e_e_i, e_e_j, ex):
    f = pl.kernel(
        _sc_pass_b,
        out_type=jax.ShapeDtypeStruct((HG * N, GW), jnp.float32),
        mesh=_SC_MESH,
        compiler_params=pltpu.CompilerParams(needs_layout_passes=False,
                                             use_tc_tiling_on_sc=False),
        scratch_types=[
            pltpu.VMEM((EPT,), jnp.int32),
            pltpu.VMEM((EPT,), jnp.int32),
            pltpu.VMEM((CH, GW), jnp.float32),
            pltpu.VMEM((CH, GW), jnp.float32),
            pltpu.VMEM((CH * HPG,), jnp.float32),
            pltpu.VMEM((CH * HPG,), jnp.float32),
            pltpu.VMEM_SHARED((N, GW), jnp.float32),
            pltpu.SemaphoreType.DMA,
            pltpu.SemaphoreType.DMA,
            pltpu.SemaphoreType.DMA,
            pltpu.SemaphoreType.DMA,
        ],
    )
    return f(vn[:N], vn[N:], e_e_i, e_e_j, ex)


def _final_body(h_ref, a0_ref, a1_ref, ln1s_ref, ln1b_ref, w_ref, b_ref,
                ln2s_ref, ln2b_ref, o_ref):
    h0 = h_ref[...]
    attn = jnp.concatenate([a0_ref[...], a1_ref[...]], axis=1)
    h = h0 + attn
    mean = jnp.mean(h, axis=-1, keepdims=True)
    var = jnp.mean((h - mean) ** 2, axis=-1, keepdims=True)
    h = (h - mean) * lax.rsqrt(var + 1e-6)
    h = h * ln1s_ref[...] + ln1b_ref[...]
    mlp = jnp.dot(h, w_ref[...], preferred_element_type=jnp.float32) + b_ref[...]
    mlp = jax.nn.gelu(mlp)
    h = h + mlp
    mean = jnp.mean(h, axis=-1, keepdims=True)
    var = jnp.mean((h - mean) ** 2, axis=-1, keepdims=True)
    h = (h - mean) * lax.rsqrt(var + 1e-6)
    o_ref[...] = h * ln2s_ref[...] + ln2b_ref[...]


def final_block(h_one, attn0, attn1, ln1_scale, ln1_bias, W_mlp, b_mlp,
                ln2_scale, ln2_bias):
    return pl.pallas_call(
        _final_body,
        grid=(N_BLKS,),
        in_specs=[
            pl.BlockSpec((ROWS_BLK, DIM), lambda i: (i, 0)),
            pl.BlockSpec((ROWS_BLK, GW), lambda i: (i, 0)),
            pl.BlockSpec((ROWS_BLK, GW), lambda i: (i, 0)),
            pl.BlockSpec((1, DIM), lambda i: (0, 0)),
            pl.BlockSpec((1, DIM), lambda i: (0, 0)),
            pl.BlockSpec((DIM, DIM), lambda i: (0, 0)),
            pl.BlockSpec((1, DIM), lambda i: (0, 0)),
            pl.BlockSpec((1, DIM), lambda i: (0, 0)),
            pl.BlockSpec((1, DIM), lambda i: (0, 0)),
        ],
        out_specs=pl.BlockSpec((ROWS_BLK, DIM), lambda i: (i, 0)),
        out_shape=jax.ShapeDtypeStruct((N, DIM), jnp.float32),
    )(h_one, attn0, attn1, ln1_scale.reshape(1, DIM), ln1_bias.reshape(1, DIM),
      W_mlp, b_mlp.reshape(1, DIM), ln2_scale.reshape(1, DIM),
      ln2_bias.reshape(1, DIM))


def kernel(h_one, e_e_i, e_e_j, W_qkv, b_qkv, ln1_scale, ln1_bias, W_mlp,
           b_mlp, ln2_scale, ln2_bias):
    e_e_i = e_e_i.astype(jnp.int32)
    e_e_j = e_e_j.astype(jnp.int32)
    q2, k2, v = qkv_project(h_one, W_qkv, b_qkv)
    ex, den = sc_pass_a(q2, k2, e_e_i, e_e_j)
    vn = vnormalize(v, den)
    attn = sc_pass_b(vn, e_e_i, e_e_j, ex)
    return final_block(h_one, attn[:N], attn[N:], ln1_scale, ln1_bias,
                       W_mlp, b_mlp, ln2_scale, ln2_bias)
